# baseline (speedup 1.0000x reference)
import jax
import jax.numpy as jnp
from jax import lax
from jax.experimental import pallas as pl
from jax.experimental.pallas import tpu as pltpu
from jax.experimental.pallas import tpu_sc as plsc

VOCAB = 1000000
HID = 64
MAXPOS = 512
TYPEV = 2
B = 1024
S = 200

NUM_CORES = 2
NUM_SUBCORES = 16
NW = NUM_CORES * NUM_SUBCORES
N_TOK = B * S
PER_W = N_TOK // NW
CHUNK = 640
N_CHUNK = PER_W // CHUNK


CTILES = B // 128
UNITS = S * CTILES
UPW = UNITS // NW


def _word_gather_body(tok_hbm, table_hbm, out_hbm, idx_v, pidx_v, rows_v,
                      face_v, sem):
    wid = lax.axis_index("s") * NUM_CORES + lax.axis_index("c")
    u0 = wid * UPW

    def unit_step(u, _):
        uid = u0 + u
        s = uid // CTILES
        c = uid % CTILES
        base = s * B + c * 128
        pltpu.sync_copy(tok_hbm.at[pl.ds(base, 128)], idx_v)
        for k in range(8):
            v = idx_v[pl.ds(k * 16, 16)]
            pidx_v[pl.ds(k * 16, 16)] = lax.shift_right_logical(v, 1)
        pltpu.async_copy(table_hbm.at[pidx_v], rows_v, sem).wait()
        for g in range(8):
            rowv = jax.lax.iota(jnp.int32, 16) + g * 16
            halfv = lax.bitwise_and(idx_v[pl.ds(g * 16, 16)], 1) * HID
            for h in range(HID):
                vals = plsc.load_gather(rows_v, [rowv, halfv + h])
                face_v[h, pl.ds(g * 16, 16)] = vals
        pltpu.sync_copy(face_v, out_hbm.at[s, :, pl.ds(c * 128, 128)])
        return _

    lax.fori_loop(0, UPW, unit_step, None)


@jax.jit
def _word_gather(tok_flat, table2):
    mesh = plsc.VectorSubcoreMesh(core_axis_name="c", subcore_axis_name="s")
    return pl.kernel(
        _word_gather_body,
        out_type=jax.ShapeDtypeStruct((S, HID, B), jnp.float32),
        mesh=mesh,
        scratch_types=[
            pltpu.VMEM((128,), jnp.int32),
            pltpu.VMEM((128,), jnp.int32),
            pltpu.VMEM((128, 128), jnp.float32),
            pltpu.VMEM((HID, 128), jnp.float32),
            pltpu.SemaphoreType.DMA,
        ],
        compiler_params=pltpu.CompilerParams(
            use_tc_tiling_on_sc=True, needs_layout_passes=False
        ),
    )(tok_flat, table2)


SS = 8


def _bcast_body(pos_ref, type_ref, pos_out, type_out):
    tbc = jnp.broadcast_to(type_ref[0], (HID, B))
    for i in range(SS):
        pos_out[i] = jnp.broadcast_to(pos_ref[i], (HID, B))
        type_out[i] = tbc


@jax.jit
def _broadcasts(w_pos_col, w_type_col):
    out_shape = jax.ShapeDtypeStruct((S, HID, B), jnp.float32)
    return pl.pallas_call(
        _bcast_body,
        grid=(S // SS,),
        in_specs=[
            pl.BlockSpec((SS, HID, 1), lambda i: (i, 0, 0)),
            pl.BlockSpec((1, HID, 1), lambda i: (0, 0, 0)),
        ],
        out_specs=[
            pl.BlockSpec((SS, HID, B), lambda i: (i, 0, 0)),
            pl.BlockSpec((SS, HID, B), lambda i: (i, 0, 0)),
        ],
        out_shape=[out_shape, out_shape],
    )(w_pos_col, w_type_col)


def kernel(token_ids, W_word, W_pos, W_type):
    tok_sb = token_ids.T.reshape(N_TOK).astype(jnp.int32)
    table2 = W_word.reshape(VOCAB // 2, 2 * HID)
    word_t = _word_gather(tok_sb, table2)
    word = word_t.transpose(2, 0, 1)
    pos_t, type_t = _broadcasts(
        W_pos[:S].reshape(S, HID, 1), W_type[0].reshape(1, HID, 1)
    )
    return (word, pos_t.transpose(2, 0, 1), type_t.transpose(2, 0, 1))


# speedup vs baseline: 2.3657x; 2.3657x over previous
import jax
import jax.numpy as jnp
from jax import lax
from jax.experimental import pallas as pl
from jax.experimental.pallas import tpu as pltpu
from jax.experimental.pallas import tpu_sc as plsc

VOCAB = 1000000
HID = 64
MAXPOS = 512
TYPEV = 2
B = 1024
S = 200

NUM_CORES = 2
NUM_SUBCORES = 16
NW = NUM_CORES * NUM_SUBCORES
N_TOK = B * S
PER_W = N_TOK // NW
CHUNK = 640
N_CHUNK = PER_W // CHUNK


def _word_gather_body(tok_hbm, table_hbm, out_hbm, idx_v, rows_v, sem):
    wid = lax.axis_index("s") * NUM_CORES + lax.axis_index("c")
    base0 = wid * PER_W

    def chunk_step(c, _):
        base = base0 + c * CHUNK
        pltpu.sync_copy(tok_hbm.at[pl.ds(base, CHUNK)], idx_v)

        def fire(g, carry):
            vec = idx_v[pl.ds(g * 16, 16)]
            for j in range(16):
                r = vec[j]
                pltpu.async_copy(
                    table_hbm.at[pl.ds(r, 1)],
                    rows_v.at[pl.ds(g * 16 + j, 1)],
                    sem,
                )
            return carry

        lax.fori_loop(0, CHUNK // 16, fire, None)
        pltpu.make_async_copy(table_hbm.at[pl.ds(0, CHUNK)], rows_v, sem).wait()
        pltpu.sync_copy(rows_v, out_hbm.at[pl.ds(base, CHUNK)])
        return _

    lax.fori_loop(0, N_CHUNK, chunk_step, None)


@jax.jit
def _word_gather(tok_flat, table_lin):
    mesh = plsc.VectorSubcoreMesh(core_axis_name="c", subcore_axis_name="s")
    return pl.kernel(
        _word_gather_body,
        out_type=jax.ShapeDtypeStruct((N_TOK, HID), jnp.float32),
        mesh=mesh,
        scratch_types=[
            pltpu.VMEM((CHUNK,), jnp.int32),
            pltpu.VMEM((CHUNK, HID), jnp.float32),
            pltpu.SemaphoreType.DMA,
        ],
        compiler_params=pltpu.CompilerParams(use_tc_tiling_on_sc=True),
    )(tok_flat, table_lin)


LB = 8192


def _repack_body(wt_ref, out_ref):
    out_ref[...] = wt_ref[...].T


@jax.jit
def _repack(w_t):
    return pl.pallas_call(
        _repack_body,
        grid=(pl.cdiv(VOCAB, LB),),
        in_specs=[pl.BlockSpec((HID, LB), lambda i: (0, i))],
        out_specs=pl.BlockSpec((LB, HID), lambda i: (i, 0)),
        out_shape=jax.ShapeDtypeStruct((VOCAB, HID), jnp.float32),
    )(w_t)


SS = 8


def _bcast_body(pos_ref, type_ref, pos_out, type_out):
    tbc = jnp.broadcast_to(type_ref[0], (HID, B))
    for i in range(SS):
        pos_out[i] = jnp.broadcast_to(pos_ref[i], (HID, B))
        type_out[i] = tbc


@jax.jit
def _broadcasts(w_pos_col, w_type_col):
    out_shape = jax.ShapeDtypeStruct((S, HID, B), jnp.float32)
    return pl.pallas_call(
        _bcast_body,
        grid=(S // SS,),
        in_specs=[
            pl.BlockSpec((SS, HID, 1), lambda i: (i, 0, 0)),
            pl.BlockSpec((1, HID, 1), lambda i: (0, 0, 0)),
        ],
        out_specs=[
            pl.BlockSpec((SS, HID, B), lambda i: (i, 0, 0)),
            pl.BlockSpec((SS, HID, B), lambda i: (i, 0, 0)),
        ],
        out_shape=[out_shape, out_shape],
    )(w_pos_col, w_type_col)


def kernel(token_ids, W_word, W_pos, W_type):
    tok_sb = token_ids.T.reshape(N_TOK).astype(jnp.int32)
    table_lin = _repack(W_word.T)
    out_rows = _word_gather(tok_sb, table_lin)
    word = out_rows.reshape(S, B, HID).transpose(1, 0, 2)
    pos_t, type_t = _broadcasts(
        W_pos[:S].reshape(S, HID, 1), W_type[0].reshape(1, HID, 1)
    )
    return (word, pos_t.transpose(2, 0, 1), type_t.transpose(2, 0, 1))


# speedup vs baseline: 2.4068x; 1.0174x over previous
import jax
import jax.numpy as jnp
from jax import lax
from jax.experimental import pallas as pl
from jax.experimental.pallas import tpu as pltpu
from jax.experimental.pallas import tpu_sc as plsc

VOCAB = 1000000
HID = 64
MAXPOS = 512
TYPEV = 2
B = 1024
S = 200

NUM_CORES = 2
NUM_SUBCORES = 16
NW = NUM_CORES * NUM_SUBCORES
N_TOK = B * S
PER_W = N_TOK // NW
CHUNK = 400
N_CHUNK = PER_W // CHUNK


def _word_gather_body(tok_hbm, table_hbm, out_hbm, idx_all, rows_a, rows_b,
                      sem_a, sem_b):
    wid = lax.axis_index("s") * NUM_CORES + lax.axis_index("c")
    base0 = wid * PER_W
    pltpu.sync_copy(tok_hbm.at[pl.ds(base0, PER_W)], idx_all)
    bufs = (rows_a, rows_b)
    sems = (sem_a, sem_b)

    def fire_chunk(c, buf, sem):
        def fire(g, carry):
            vec = idx_all[pl.ds(c * CHUNK + g * 16, 16)]
            for j in range(16):
                pltpu.async_copy(
                    table_hbm.at[pl.ds(vec[j], 1)],
                    buf.at[pl.ds(g * 16 + j, 1)],
                    sem,
                )
            return carry

        lax.fori_loop(0, CHUNK // 16, fire, None)

    fire_chunk(0, bufs[0], sems[0])
    for c in range(N_CHUNK):
        if c + 1 < N_CHUNK:
            fire_chunk(c + 1, bufs[(c + 1) % 2], sems[(c + 1) % 2])
        pltpu.make_async_copy(
            table_hbm.at[pl.ds(0, CHUNK)], bufs[c % 2], sems[c % 2]
        ).wait()
        pltpu.sync_copy(bufs[c % 2], out_hbm.at[pl.ds(base0 + c * CHUNK, CHUNK)])


@jax.jit
def _word_gather(tok_flat, table_lin):
    mesh = plsc.VectorSubcoreMesh(core_axis_name="c", subcore_axis_name="s")
    return pl.kernel(
        _word_gather_body,
        out_type=jax.ShapeDtypeStruct((N_TOK, HID), jnp.float32),
        mesh=mesh,
        scratch_types=[
            pltpu.VMEM((PER_W,), jnp.int32),
            pltpu.VMEM((CHUNK, HID), jnp.float32),
            pltpu.VMEM((CHUNK, HID), jnp.float32),
            pltpu.SemaphoreType.DMA,
            pltpu.SemaphoreType.DMA,
        ],
        compiler_params=pltpu.CompilerParams(use_tc_tiling_on_sc=True),
    )(tok_flat, table_lin)


LB = 8192


def _repack_body(wt_ref, out_ref):
    out_ref[...] = wt_ref[...].T


@jax.jit
def _repack(w_t):
    return pl.pallas_call(
        _repack_body,
        grid=(pl.cdiv(VOCAB, LB),),
        in_specs=[pl.BlockSpec((HID, LB), lambda i: (0, i))],
        out_specs=pl.BlockSpec((LB, HID), lambda i: (i, 0)),
        out_shape=jax.ShapeDtypeStruct((VOCAB, HID), jnp.float32),
    )(w_t)


SS = 20


def _bcast_body(pos_ref, type_ref, pos_out, type_out):
    tbc = jnp.broadcast_to(type_ref[0], (HID, B))
    for i in range(SS):
        pos_out[i] = jnp.broadcast_to(pos_ref[i], (HID, B))
        type_out[i] = tbc


@jax.jit
def _broadcasts(w_pos_col, w_type_col):
    out_shape = jax.ShapeDtypeStruct((S, HID, B), jnp.float32)
    return pl.pallas_call(
        _bcast_body,
        grid=(S // SS,),
        in_specs=[
            pl.BlockSpec((SS, HID, 1), lambda i: (i, 0, 0)),
            pl.BlockSpec((1, HID, 1), lambda i: (0, 0, 0)),
        ],
        out_specs=[
            pl.BlockSpec((SS, HID, B), lambda i: (i, 0, 0)),
            pl.BlockSpec((SS, HID, B), lambda i: (i, 0, 0)),
        ],
        out_shape=[out_shape, out_shape],
    )(w_pos_col, w_type_col)


def kernel(token_ids, W_word, W_pos, W_type):
    tok_sb = token_ids.T.reshape(N_TOK).astype(jnp.int32)
    table_lin = _repack(W_word.T)
    out_rows = _word_gather(tok_sb, table_lin)
    word = out_rows.reshape(S, B, HID).transpose(1, 0, 2)
    pos_t, type_t = _broadcasts(
        W_pos[:S].reshape(S, HID, 1), W_type[0].reshape(1, HID, 1)
    )
    return (word, pos_t.transpose(2, 0, 1), type_t.transpose(2, 0, 1))


# speedup vs baseline: 2.5234x; 1.0484x over previous
import jax
import jax.numpy as jnp
from jax import lax
from jax.experimental import pallas as pl
from jax.experimental.pallas import tpu as pltpu
from jax.experimental.pallas import tpu_sc as plsc

VOCAB = 1000000
HID = 64
MAXPOS = 512
TYPEV = 2
B = 1024
S = 200

NUM_CORES = 2
NUM_SUBCORES = 16
NW = NUM_CORES * NUM_SUBCORES
N_TOK = B * S
PER_W = N_TOK // NW
CHUNK = 400
N_CHUNK = PER_W // CHUNK


def _word_gather_body(tok_hbm, table_hbm, out_hbm, idx_all, rows_a, rows_b,
                      sem_a, sem_b):
    wid = lax.axis_index("s") * NUM_CORES + lax.axis_index("c")
    base0 = wid * PER_W
    pltpu.sync_copy(tok_hbm.at[pl.ds(base0, PER_W)], idx_all)
    bufs = (rows_a, rows_b)
    sems = (sem_a, sem_b)

    def fire_chunk(c, buf, sem):
        def fire(g, carry):
            vec = idx_all[pl.ds(c * CHUNK + g * 16, 16)]
            for j in range(16):
                pltpu.async_copy(
                    table_hbm.at[pl.ds(vec[j], 1)],
                    buf.at[pl.ds(g * 16 + j, 1)],
                    sem,
                )
            return carry

        lax.fori_loop(0, CHUNK // 16, fire, None)

    fire_chunk(0, bufs[0], sems[0])
    for c in range(N_CHUNK):
        if c + 1 < N_CHUNK:
            fire_chunk(c + 1, bufs[(c + 1) % 2], sems[(c + 1) % 2])
        pltpu.make_async_copy(
            table_hbm.at[pl.ds(0, CHUNK)], bufs[c % 2], sems[c % 2]
        ).wait()
        pltpu.sync_copy(bufs[c % 2], out_hbm.at[pl.ds(base0 + c * CHUNK, CHUNK)])


@jax.jit
def _word_gather(tok_flat, table_lin):
    mesh = plsc.VectorSubcoreMesh(core_axis_name="c", subcore_axis_name="s")
    return pl.kernel(
        _word_gather_body,
        out_type=jax.ShapeDtypeStruct((N_TOK, HID), jnp.float32),
        mesh=mesh,
        scratch_types=[
            pltpu.VMEM((PER_W,), jnp.int32),
            pltpu.VMEM((CHUNK, HID), jnp.float32),
            pltpu.VMEM((CHUNK, HID), jnp.float32),
            pltpu.SemaphoreType.DMA,
            pltpu.SemaphoreType.DMA,
        ],
        compiler_params=pltpu.CompilerParams(use_tc_tiling_on_sc=True),
    )(tok_flat, table_lin)


LB = 16384


def _repack_body(wt_ref, out_ref):
    out_ref[...] = wt_ref[...].T


@jax.jit
def _repack(w_t):
    return pl.pallas_call(
        _repack_body,
        grid=(pl.cdiv(VOCAB, LB),),
        in_specs=[pl.BlockSpec((HID, LB), lambda i: (0, i))],
        out_specs=pl.BlockSpec((LB, HID), lambda i: (i, 0)),
        out_shape=jax.ShapeDtypeStruct((VOCAB, HID), jnp.float32),
    )(w_t)


SS = 20


def _bcast_body(pos_ref, type_ref, pos_out, type_out):
    tbc = jnp.broadcast_to(type_ref[0], (HID, B))
    for i in range(SS):
        pos_out[i] = jnp.broadcast_to(pos_ref[i], (HID, B))
        type_out[i] = tbc


@jax.jit
def _broadcasts(w_pos_col, w_type_col):
    out_shape = jax.ShapeDtypeStruct((S, HID, B), jnp.float32)
    return pl.pallas_call(
        _bcast_body,
        grid=(S // SS,),
        in_specs=[
            pl.BlockSpec((SS, HID, 1), lambda i: (i, 0, 0)),
            pl.BlockSpec((1, HID, 1), lambda i: (0, 0, 0)),
        ],
        out_specs=[
            pl.BlockSpec((SS, HID, B), lambda i: (i, 0, 0)),
            pl.BlockSpec((SS, HID, B), lambda i: (i, 0, 0)),
        ],
        out_shape=[out_shape, out_shape],
    )(w_pos_col, w_type_col)


def kernel(token_ids, W_word, W_pos, W_type):
    tok_sb = token_ids.T.reshape(N_TOK).astype(jnp.int32)
    table_lin = _repack(W_word.T)
    pos_t, type_t = _broadcasts(
        W_pos[:S].reshape(S, HID, 1), W_type[0].reshape(1, HID, 1)
    )
    out_rows = _word_gather(tok_sb, table_lin)
    word = out_rows.reshape(S, B, HID).transpose(1, 0, 2)
    return (word, pos_t.transpose(2, 0, 1), type_t.transpose(2, 0, 1))


# speedup vs baseline: 2.5250x; 1.0007x over previous
import jax
import jax.numpy as jnp
from jax import lax
from jax.experimental import pallas as pl
from jax.experimental.pallas import tpu as pltpu
from jax.experimental.pallas import tpu_sc as plsc

VOCAB = 1000000
HID = 64
MAXPOS = 512
TYPEV = 2
B = 1024
S = 200

NUM_CORES = 2
NUM_SUBCORES = 16
NW = NUM_CORES * NUM_SUBCORES
N_TOK = B * S
PER_W = N_TOK // NW
CHUNK = 400
N_CHUNK = PER_W // CHUNK


def _word_gather_body(tok_hbm, table_hbm, out_hbm, idx_all, rows_a, rows_b,
                      sem_a, sem_b):
    wid = lax.axis_index("s") * NUM_CORES + lax.axis_index("c")
    base0 = wid * PER_W
    pltpu.sync_copy(tok_hbm.at[pl.ds(base0, PER_W)], idx_all)
    bufs = (rows_a, rows_b)
    sems = (sem_a, sem_b)

    def fire_chunk(c, buf, sem):
        def fire(g, carry):
            vec = idx_all[pl.ds(c * CHUNK + g * 16, 16)]
            for j in range(16):
                pltpu.async_copy(
                    table_hbm.at[pl.ds(vec[j], 1)],
                    buf.at[pl.ds(g * 16 + j, 1)],
                    sem,
                )
            return carry

        lax.fori_loop(0, CHUNK // 16, fire, None)

    fire_chunk(0, bufs[0], sems[0])
    for c in range(N_CHUNK):
        if c + 1 < N_CHUNK:
            fire_chunk(c + 1, bufs[(c + 1) % 2], sems[(c + 1) % 2])
        pltpu.make_async_copy(
            table_hbm.at[pl.ds(0, CHUNK)], bufs[c % 2], sems[c % 2]
        ).wait()
        pltpu.sync_copy(bufs[c % 2], out_hbm.at[pl.ds(base0 + c * CHUNK, CHUNK)])


@jax.jit
def _word_gather(tok_flat, table_lin):
    mesh = plsc.VectorSubcoreMesh(core_axis_name="c", subcore_axis_name="s")
    return pl.kernel(
        _word_gather_body,
        out_type=jax.ShapeDtypeStruct((N_TOK, HID), jnp.float32),
        mesh=mesh,
        scratch_types=[
            pltpu.VMEM((PER_W,), jnp.int32),
            pltpu.VMEM((CHUNK, HID), jnp.float32),
            pltpu.VMEM((CHUNK, HID), jnp.float32),
            pltpu.SemaphoreType.DMA,
            pltpu.SemaphoreType.DMA,
        ],
        compiler_params=pltpu.CompilerParams(use_tc_tiling_on_sc=True),
        cost_estimate=pl.CostEstimate(
            flops=0, bytes_accessed=160_000_000, transcendentals=0
        ),
    )(tok_flat, table_lin)


LB = 16384


def _repack_body(wt_ref, out_ref):
    out_ref[...] = wt_ref[...].T


@jax.jit
def _repack(w_t):
    return pl.pallas_call(
        _repack_body,
        grid=(pl.cdiv(VOCAB, LB),),
        in_specs=[pl.BlockSpec((HID, LB), lambda i: (0, i))],
        out_specs=pl.BlockSpec((LB, HID), lambda i: (i, 0)),
        out_shape=jax.ShapeDtypeStruct((VOCAB, HID), jnp.float32),
    )(w_t)


SS = 20


def _bcast_body(pos_ref, type_ref, pos_out, type_out):
    tbc = jnp.broadcast_to(type_ref[0], (HID, B))
    for i in range(SS):
        pos_out[i] = jnp.broadcast_to(pos_ref[i], (HID, B))
        type_out[i] = tbc


@jax.jit
def _broadcasts(w_pos_col, w_type_col):
    out_shape = jax.ShapeDtypeStruct((S, HID, B), jnp.float32)
    return pl.pallas_call(
        _bcast_body,
        grid=(S // SS,),
        in_specs=[
            pl.BlockSpec((SS, HID, 1), lambda i: (i, 0, 0)),
            pl.BlockSpec((1, HID, 1), lambda i: (0, 0, 0)),
        ],
        out_specs=[
            pl.BlockSpec((SS, HID, B), lambda i: (i, 0, 0)),
            pl.BlockSpec((SS, HID, B), lambda i: (i, 0, 0)),
        ],
        out_shape=[out_shape, out_shape],
    )(w_pos_col, w_type_col)


def kernel(token_ids, W_word, W_pos, W_type):
    tok_sb = token_ids.T.reshape(N_TOK).astype(jnp.int32)
    table_lin = _repack(W_word.T)
    pos_t, type_t = _broadcasts(
        W_pos[:S].reshape(S, HID, 1), W_type[0].reshape(1, HID, 1)
    )
    out_rows = _word_gather(tok_sb, table_lin)
    word = out_rows.reshape(S, B, HID).transpose(1, 0, 2)
    return (word, pos_t.transpose(2, 0, 1), type_t.transpose(2, 0, 1))
